# baseline (speedup 1.0000x reference)
import jax
import jax.numpy as jnp
from jax import lax
from jax.experimental import pallas as pl
from jax.experimental.pallas import tpu as pltpu
from jax.experimental.pallas import tpu_sc as plsc

N = 10000
FANOUT = 32
D_FEAT = 128
OUT_DIM = 128
LANES = 16
VPF = D_FEAT // LANES

NUM_WORKERS = 32
N_SC = 4096
N_TC = N - N_SC
NODES_PER_WORKER = N_SC // NUM_WORKERS
GROUP = 16
N_GROUPS = NODES_PER_WORKER // GROUP

BLOCK_TC = 328
BLOCK_EPI = 512


def _tc_body(src_ref, dst_ref, w1_ref, w2_ref, b_ref, out_ref):
    agg = jnp.mean(dst_ref[...], axis=1)
    out_ref[...] = (
        jnp.dot(src_ref[...], w1_ref[...], preferred_element_type=jnp.float32)
        + jnp.dot(agg, w2_ref[...], preferred_element_type=jnp.float32)
        + b_ref[0:1, :]
    )


def _tc_fused(src, dst, w1, w2, b2d, rows):
    grid = (rows // BLOCK_TC,)
    return pl.pallas_call(
        _tc_body,
        grid=grid,
        in_specs=[
            pl.BlockSpec((BLOCK_TC, D_FEAT), lambda i: (i, 0)),
            pl.BlockSpec((BLOCK_TC, FANOUT, D_FEAT), lambda i: (i, 0, 0)),
            pl.BlockSpec((D_FEAT, OUT_DIM), lambda i: (0, 0)),
            pl.BlockSpec((D_FEAT, OUT_DIM), lambda i: (0, 0)),
            pl.BlockSpec((8, OUT_DIM), lambda i: (0, 0)),
        ],
        out_specs=pl.BlockSpec((BLOCK_TC, OUT_DIM), lambda i: (i, 0)),
        out_shape=jax.ShapeDtypeStruct((rows, OUT_DIM), jnp.float32),
    )(src, dst, w1, w2, b2d)


def _tc_epi_body(src_ref, sum_ref, w1_ref, w2s_ref, b_ref, out_ref):
    out_ref[...] = (
        jnp.dot(src_ref[...], w1_ref[...], preferred_element_type=jnp.float32)
        + jnp.dot(sum_ref[...], w2s_ref[...], preferred_element_type=jnp.float32)
        + b_ref[0:1, :]
    )


def _tc_epilogue(src, sums, w1, w2s, b2d, rows):
    grid = (rows // BLOCK_EPI,)
    return pl.pallas_call(
        _tc_epi_body,
        grid=grid,
        in_specs=[
            pl.BlockSpec((BLOCK_EPI, D_FEAT), lambda i: (i, 0)),
            pl.BlockSpec((BLOCK_EPI, D_FEAT), lambda i: (i, 0)),
            pl.BlockSpec((D_FEAT, OUT_DIM), lambda i: (0, 0)),
            pl.BlockSpec((D_FEAT, OUT_DIM), lambda i: (0, 0)),
            pl.BlockSpec((8, OUT_DIM), lambda i: (0, 0)),
        ],
        out_specs=pl.BlockSpec((BLOCK_EPI, OUT_DIM), lambda i: (i, 0)),
        out_shape=jax.ShapeDtypeStruct((rows, OUT_DIM), jnp.float32),
    )(src, sums, w1, w2s, b2d)


ROW_ELEMS = FANOUT * D_FEAT
NPC = N_SC // 2
CH = 128
NCHUNK = NPC // CH
N_ISS = 8
NPI = CH // N_ISS
CPT = CH // 16
CH_ELEMS = CH * ROW_ELEMS


def _sc_reduce_tile(buf, acc):
    def node(n, _):
        b = n * ROW_ELEMS
        accs = [buf[pl.ds(b + j * LANES, LANES)] for j in range(VPF)]
        for k in range(1, FANOUT):
            for j in range(VPF):
                accs[j] = accs[j] + buf[pl.ds(b + k * D_FEAT + j * LANES, LANES)]
        for j in range(VPF):
            acc[pl.ds(n * D_FEAT + j * LANES, LANES)] = accs[j]
        return ()

    lax.fori_loop(0, CPT, node, ())


def _sc_sum_body(dst_hbm, sum_hbm, spbuf, tilebuf, acc, sem_a, sem_b, psem):
    c = lax.axis_index("c")
    s = lax.axis_index("s")

    def stage_desc(q, slot, sem):
        start = (N_TC + c * NPC) * ROW_ELEMS + q * CH_ELEMS + s * NPI * ROW_ELEMS
        dst = spbuf.at[pl.ds(slot * CH_ELEMS + s * NPI * ROW_ELEMS, NPI * ROW_ELEMS)]
        return pltpu.make_async_copy(dst_hbm.at[pl.ds(start, NPI * ROW_ELEMS)], dst, sem)

    def half(q, slot, sem):
        @pl.when(s < N_ISS)
        def _():
            stage_desc(q, slot, sem).wait()

        plsc.subcore_barrier()

        pull = pltpu.make_async_copy(
            spbuf.at[pl.ds(slot * CH_ELEMS + s * CPT * ROW_ELEMS, CPT * ROW_ELEMS)],
            tilebuf,
            psem,
        )
        pull.start()
        pull.wait()
        _sc_reduce_tile(tilebuf, acc)
        out_off = (c * NPC + q * CH) * D_FEAT + s * CPT * D_FEAT
        pltpu.sync_copy(acc, sum_hbm.at[pl.ds(out_off, CPT * D_FEAT)])
        plsc.subcore_barrier()

        @pl.when((s < N_ISS) & (q + 2 < NCHUNK))
        def _():
            stage_desc(q + 2, slot, sem).start()

    @pl.when(s < N_ISS)
    def _():
        stage_desc(0, 0, sem_a).start()
        stage_desc(1, 1, sem_b).start()

    def pair(i, _):
        q = 2 * i
        half(q, 0, sem_a)
        half(q + 1, 1, sem_b)
        return ()

    lax.fori_loop(0, NCHUNK // 2, pair, ())


def _sc_sums(dst_feature):
    kern = pl.kernel(
        _sc_sum_body,
        out_type=jax.ShapeDtypeStruct((N_SC * D_FEAT,), jnp.float32),
        mesh=plsc.VectorSubcoreMesh(core_axis_name="c", subcore_axis_name="s"),
        scratch_types=[
            pltpu.VMEM_SHARED((2 * CH_ELEMS,), jnp.float32),
            pltpu.VMEM((CPT * ROW_ELEMS,), jnp.float32),
            pltpu.VMEM((CPT * D_FEAT,), jnp.float32),
            pltpu.SemaphoreType.DMA,
            pltpu.SemaphoreType.DMA,
            pltpu.SemaphoreType.DMA,
        ],
    )
    return kern(dst_feature.reshape(-1)).reshape(N_SC, D_FEAT)


def kernel(src_feature, dst_feature, W, b):
    w1 = W[:D_FEAT]
    w2 = W[D_FEAT:]
    w2s = w2 * (1.0 / FANOUT)
    b2d = jnp.broadcast_to(b.reshape(1, OUT_DIM), (8, OUT_DIM))

    sums_sc = _sc_sums(dst_feature)
    out_tc = _tc_fused(src_feature, dst_feature, w1, w2, b2d, N_TC)
    out_sc = _tc_epilogue(src_feature[N_TC:], sums_sc, w1, w2s, b2d, N_SC)
    return jnp.concatenate([out_tc, out_sc], axis=0)


# speedup vs baseline: 1.1403x; 1.1403x over previous
import jax
import jax.numpy as jnp
from jax import lax
from jax.experimental import pallas as pl
from jax.experimental.pallas import tpu as pltpu
from jax.experimental.pallas import tpu_sc as plsc

N = 10000
FANOUT = 32
D_FEAT = 128
OUT_DIM = 128
LANES = 16
VPF = D_FEAT // LANES

NUM_WORKERS = 32
N_SC = 3072
N_TC = N - N_SC

BLOCK_TC = 400
N_TC_PAD = -(-N_TC // BLOCK_TC) * BLOCK_TC
BLOCK_EPI = 512


def _tc_body(src_ref, dst_ref, w1_ref, w2_ref, b_ref, out_ref):
    agg = jnp.mean(dst_ref[...], axis=1)
    out_ref[...] = (
        jnp.dot(src_ref[...], w1_ref[...], preferred_element_type=jnp.float32)
        + jnp.dot(agg, w2_ref[...], preferred_element_type=jnp.float32)
        + b_ref[0:1, :]
    )


def _tc_fused(src, dst, w1, w2, b2d, rows):
    grid = (-(-rows // BLOCK_TC),)
    return pl.pallas_call(
        _tc_body,
        grid=grid,
        in_specs=[
            pl.BlockSpec((BLOCK_TC, D_FEAT), lambda i: (i, 0)),
            pl.BlockSpec((BLOCK_TC, FANOUT, D_FEAT), lambda i: (i, 0, 0)),
            pl.BlockSpec((D_FEAT, OUT_DIM), lambda i: (0, 0)),
            pl.BlockSpec((D_FEAT, OUT_DIM), lambda i: (0, 0)),
            pl.BlockSpec((8, OUT_DIM), lambda i: (0, 0)),
        ],
        out_specs=pl.BlockSpec((BLOCK_TC, OUT_DIM), lambda i: (i, 0)),
        out_shape=jax.ShapeDtypeStruct((N_TC_PAD, OUT_DIM), jnp.float32),
    )(src, dst, w1, w2, b2d)


def _tc_epi_body(src_ref, sum_ref, w1_ref, w2s_ref, b_ref, out_ref):
    out_ref[...] = (
        jnp.dot(src_ref[...], w1_ref[...], preferred_element_type=jnp.float32)
        + jnp.dot(sum_ref[...], w2s_ref[...], preferred_element_type=jnp.float32)
        + b_ref[0:1, :]
    )


def _tc_epilogue(src, sums, w1, w2s, b2d, rows):
    grid = (rows // BLOCK_EPI,)
    return pl.pallas_call(
        _tc_epi_body,
        grid=grid,
        in_specs=[
            pl.BlockSpec((BLOCK_EPI, D_FEAT), lambda i: (i, 0)),
            pl.BlockSpec((BLOCK_EPI, D_FEAT), lambda i: (i, 0)),
            pl.BlockSpec((D_FEAT, OUT_DIM), lambda i: (0, 0)),
            pl.BlockSpec((D_FEAT, OUT_DIM), lambda i: (0, 0)),
            pl.BlockSpec((8, OUT_DIM), lambda i: (0, 0)),
        ],
        out_specs=pl.BlockSpec((BLOCK_EPI, OUT_DIM), lambda i: (i, 0)),
        out_shape=jax.ShapeDtypeStruct((rows, OUT_DIM), jnp.float32),
    )(src, sums, w1, w2s, b2d)


ROW_ELEMS = FANOUT * D_FEAT
NPC = N_SC // 2
CH = 128
NCHUNK = NPC // CH
N_ISS = 8
NPI = CH // N_ISS
CPT = CH // 16
CPH = CPT // 2
CH_ELEMS = CH * ROW_ELEMS


def _sc_reduce_tile(buf, acc, acc_base):
    def node(n, _):
        b = n * ROW_ELEMS
        accs = [buf[pl.ds(b + j * LANES, LANES)] for j in range(VPF)]
        for k in range(1, FANOUT):
            for j in range(VPF):
                accs[j] = accs[j] + buf[pl.ds(b + k * D_FEAT + j * LANES, LANES)]
        for j in range(VPF):
            acc[pl.ds(acc_base + n * D_FEAT + j * LANES, LANES)] = accs[j]
        return ()

    lax.fori_loop(0, CPH, node, ())


def _sc_sum_body(dst_hbm, sum_hbm, spbuf, tb0, tb1, acc, sem_a, sem_b, psem0, psem1):
    c = lax.axis_index("c")
    s = lax.axis_index("s")

    def stage_desc(q, slot, sem):
        start = (N_TC + c * NPC) * ROW_ELEMS + q * CH_ELEMS + s * NPI * ROW_ELEMS
        dst = spbuf.at[pl.ds(slot * CH_ELEMS + s * NPI * ROW_ELEMS, NPI * ROW_ELEMS)]
        return pltpu.make_async_copy(dst_hbm.at[pl.ds(start, NPI * ROW_ELEMS)], dst, sem)

    def half(q, slot, sem):
        @pl.when(s < N_ISS)
        def _():
            stage_desc(q, slot, sem).wait()

        plsc.subcore_barrier()

        tile_base = slot * CH_ELEMS + s * CPT * ROW_ELEMS
        p0 = pltpu.make_async_copy(
            spbuf.at[pl.ds(tile_base, CPH * ROW_ELEMS)], tb0, psem0)
        p1 = pltpu.make_async_copy(
            spbuf.at[pl.ds(tile_base + CPH * ROW_ELEMS, CPH * ROW_ELEMS)], tb1, psem1)
        p0.start()
        p1.start()
        p0.wait()
        _sc_reduce_tile(tb0, acc, 0)
        p1.wait()
        _sc_reduce_tile(tb1, acc, CPH * D_FEAT)
        out_off = (c * NPC + q * CH) * D_FEAT + s * CPT * D_FEAT
        pltpu.sync_copy(acc, sum_hbm.at[pl.ds(out_off, CPT * D_FEAT)])
        plsc.subcore_barrier()

        @pl.when((s < N_ISS) & (q + 2 < NCHUNK))
        def _():
            stage_desc(q + 2, slot, sem).start()

    @pl.when(s < N_ISS)
    def _():
        stage_desc(0, 0, sem_a).start()
        stage_desc(1, 1, sem_b).start()

    def pair(i, _):
        q = 2 * i
        half(q, 0, sem_a)
        half(q + 1, 1, sem_b)
        return ()

    lax.fori_loop(0, NCHUNK // 2, pair, ())


def _sc_sums(dst_feature):
    kern = pl.kernel(
        _sc_sum_body,
        out_type=jax.ShapeDtypeStruct((N_SC * D_FEAT,), jnp.float32),
        mesh=plsc.VectorSubcoreMesh(core_axis_name="c", subcore_axis_name="s"),
        scratch_types=[
            pltpu.VMEM_SHARED((2 * CH_ELEMS,), jnp.float32),
            pltpu.VMEM((CPH * ROW_ELEMS,), jnp.float32),
            pltpu.VMEM((CPH * ROW_ELEMS,), jnp.float32),
            pltpu.VMEM((CPT * D_FEAT,), jnp.float32),
            pltpu.SemaphoreType.DMA,
            pltpu.SemaphoreType.DMA,
            pltpu.SemaphoreType.DMA,
            pltpu.SemaphoreType.DMA,
        ],
    )
    return kern(dst_feature.reshape(-1)).reshape(N_SC, D_FEAT)


def kernel(src_feature, dst_feature, W, b):
    w1 = W[:D_FEAT]
    w2 = W[D_FEAT:]
    w2s = w2 * (1.0 / FANOUT)
    b2d = jnp.broadcast_to(b.reshape(1, OUT_DIM), (8, OUT_DIM))

    sums_sc = _sc_sums(dst_feature)
    out_tc = _tc_fused(src_feature, dst_feature, w1, w2, b2d, N_TC)
    out_sc = _tc_epilogue(src_feature[N_TC:], sums_sc, w1, w2s, b2d, N_SC)
    return jnp.concatenate([out_tc[:N_TC], out_sc], axis=0)


# speedup vs baseline: 1.7680x; 1.5504x over previous
import jax
import jax.numpy as jnp
from jax.experimental import pallas as pl

N = 10000
FANOUT = 32
D_FEAT = 128
OUT_DIM = 128
BLOCK_ROWS = 400


def _body(src_ref, dst_ref, w1_ref, w2_ref, b_ref, out_ref):
    agg = jnp.mean(dst_ref[...], axis=1)
    out_ref[...] = (
        jnp.dot(src_ref[...], w1_ref[...], preferred_element_type=jnp.float32)
        + jnp.dot(agg, w2_ref[...], preferred_element_type=jnp.float32)
        + b_ref[0:1, :]
    )


def kernel(src_feature, dst_feature, W, b):
    w1 = W[:D_FEAT]
    w2 = W[D_FEAT:]
    b2d = jnp.broadcast_to(b.reshape(1, OUT_DIM), (8, OUT_DIM))
    grid = (N // BLOCK_ROWS,)
    return pl.pallas_call(
        _body,
        grid=grid,
        in_specs=[
            pl.BlockSpec((BLOCK_ROWS, D_FEAT), lambda i: (i, 0)),
            pl.BlockSpec((BLOCK_ROWS, FANOUT, D_FEAT), lambda i: (i, 0, 0)),
            pl.BlockSpec((D_FEAT, OUT_DIM), lambda i: (0, 0)),
            pl.BlockSpec((D_FEAT, OUT_DIM), lambda i: (0, 0)),
            pl.BlockSpec((8, OUT_DIM), lambda i: (0, 0)),
        ],
        out_specs=pl.BlockSpec((BLOCK_ROWS, OUT_DIM), lambda i: (i, 0)),
        out_shape=jax.ShapeDtypeStruct((N, OUT_DIM), jnp.float32),
    )(src_feature, dst_feature, w1, w2, b2d)
